# baseline (speedup 1.0000x reference)
import functools

import jax
import jax.numpy as jnp
from jax import lax
from jax.experimental import pallas as pl
from jax.experimental.pallas import tpu as pltpu
from jax.experimental.pallas import tpu_sc as plsc

_VOCAB = 1000000
_D = 32
_BATCH = 4096
_HIST = 200
_N = _BATCH * _HIST
_NC, _NS = 2, 16
_NW = _NC * _NS
_BTG = 2
_CB = 128 * _BTG
_NBLK = _N // _CB
_PER_W = _NBLK // _NW
_BPH = _BATCH // _CB
_NBUF = 4


def _make_gather():
  mesh = plsc.VectorSubcoreMesh(
      core_axis_name="c", subcore_axis_name="s",
      num_cores=_NC, num_subcores=_NS)

  @functools.partial(
      pl.kernel,
      mesh=mesh,
      out_type=jax.ShapeDtypeStruct((_HIST, 4, _BATCH * 8), jnp.float32),
      scratch_types=[
          pltpu.VMEM((_PER_W * _CB,), jnp.int32),
          pltpu.VMEM((_NBUF, _CB, _D), jnp.float32),
          pltpu.VMEM((_NBUF, _CB * _D), jnp.float32),
          pltpu.VMEM((_CB * 16,), jnp.int32),
          pltpu.SemaphoreType.DMA,
          [pltpu.SemaphoreType.DMA] * _NBUF,
          [pltpu.SemaphoreType.DMA] * _NBUF,
      ],
      compiler_params=pltpu.CompilerParams(
          use_tc_tiling_on_sc=False, needs_layout_passes=False),
  )
  def gather(doc_hbm, tab_hbm, out_hbm, idx_v, rows_v, trans_v, ilo_v,
             isem, gsems, wsems):
    wid = lax.axis_index("s") * _NC + lax.axis_index("c")
    base_blk = wid * _PER_W

    lanes = lax.iota(jnp.int32, 16)
    base0 = jnp.where(lanes < 8, lanes * 128,
                      _CB * 8 + (lanes - 8) * 128)

    def blk_coords(k):
      bid = base_blk + k
      return bid // _BPH, (bid % _BPH) * _CB

    def start_gather(k, s):
      pltpu.async_copy(tab_hbm.at[idx_v.at[pl.ds(k * _CB, _CB)]],
                       rows_v.at[s], gsems[s])

    def wait_gather(s):
      pltpu.make_async_copy(tab_hbm.at[idx_v.at[pl.ds(0, _CB)]],
                            rows_v.at[s], gsems[s]).wait()

    def start_writes(k, s):
      h, off = blk_coords(k)
      for ft in range(4):
        pltpu.async_copy(
            trans_v.at[s, pl.ds(ft * _CB * 8, _CB * 8)],
            out_hbm.at[h, ft, pl.ds(off * 8, _CB * 8)], wsems[s])

    def wait_writes(s):
      for ft in range(4):
        pltpu.make_async_copy(
            trans_v.at[s, pl.ds(ft * _CB * 8, _CB * 8)],
            out_hbm.at[0, 0, pl.ds(0, _CB * 8)], wsems[s]).wait()

    @plsc.parallel_loop(0, _CB, unroll=8)
    def _(r):
      ilo_v[pl.ds(r * 16, 16)] = base0 + ((r >> 7) * 1024 + (r & 127))

    def transpose(s):
      @plsc.parallel_loop(0, _CB, unroll=8)
      def _(r):
        ilo = ilo_v[pl.ds(r * 16, 16)]
        plsc.store_scatter(trans_v.at[s], [ilo],
                           rows_v[s, r, pl.ds(0, 16)])
        plsc.store_scatter(trans_v.at[s], [ilo + _CB * 16],
                           rows_v[s, r, pl.ds(16, 16)])

    pltpu.async_copy(doc_hbm.at[pl.ds(base_blk * _CB, _PER_W * _CB)],
                     idx_v, isem).wait()
    for s in range(_NBUF):
      start_gather(s, s)

    def body(g, carry):
      for s in range(_NBUF):
        k = _NBUF * g + s
        wait_gather(s)

        @pl.when(g >= 1)
        def _():
          wait_writes(s)

        transpose(s)
        start_writes(k, s)

        @pl.when(g < _PER_W // _NBUF - 1)
        def _():
          start_gather(k + _NBUF, s)
      return carry

    lax.fori_loop(0, _PER_W // _NBUF, body, 0)
    for s in range(_NBUF):
      wait_writes(s)

  return gather


_gather = _make_gather()


def kernel(doc, table):
  doc_flat = doc.T.reshape(_N)
  tab_flat = lax.optimization_barrier(table.reshape(_VOCAB * _D))
  tab2 = tab_flat.reshape(_VOCAB, _D)
  out5 = _gather(doc_flat, tab2)
  out6 = out5.reshape(_HIST, 4, _BATCH // 128, 8, 128)
  return out6.transpose(2, 4, 0, 1, 3).reshape(_BATCH, _HIST, _D)


# speedup vs baseline: 1.0170x; 1.0170x over previous
import functools

import jax
import jax.numpy as jnp
from jax import lax
from jax.experimental import pallas as pl
from jax.experimental.pallas import tpu as pltpu
from jax.experimental.pallas import tpu_sc as plsc

_VOCAB = 1000000
_D = 32
_BATCH = 4096
_HIST = 200
_N = _BATCH * _HIST
_NC, _NS = 2, 16
_NW = _NC * _NS
_BTG = 2
_CB = 128 * _BTG
_NBLK = _N // _CB
_PER_W = _NBLK // _NW
_BPH = _BATCH // _CB
_NBUF = 4


def _make_gather():
  mesh = plsc.VectorSubcoreMesh(
      core_axis_name="c", subcore_axis_name="s",
      num_cores=_NC, num_subcores=_NS)

  @functools.partial(
      pl.kernel,
      mesh=mesh,
      out_type=jax.ShapeDtypeStruct((_HIST, 4, _BATCH * 8), jnp.float32),
      scratch_types=[
          pltpu.VMEM((_PER_W * _CB,), jnp.int32),
          pltpu.VMEM((_NBUF, _CB, _D), jnp.float32),
          pltpu.VMEM((_NBUF, _CB * _D), jnp.float32),
          pltpu.SemaphoreType.DMA,
          [pltpu.SemaphoreType.DMA] * _NBUF,
          [pltpu.SemaphoreType.DMA] * _NBUF,
      ],
      compiler_params=pltpu.CompilerParams(
          use_tc_tiling_on_sc=False, needs_layout_passes=False),
  )
  def gather(doc_hbm, tab_hbm, out_hbm, idx_v, rows_v, trans_v, isem,
             gsems, wsems):
    wid = lax.axis_index("s") * _NC + lax.axis_index("c")
    base_blk = wid * _PER_W

    lanes = lax.iota(jnp.int32, 16)
    base0 = jnp.where(lanes < 8, lanes * 128,
                      _CB * 8 + (lanes - 8) * 128)

    def blk_coords(k):
      bid = base_blk + k
      return bid // _BPH, (bid % _BPH) * _CB

    def start_gather(k, s):
      pltpu.async_copy(tab_hbm.at[idx_v.at[pl.ds(k * _CB, _CB)]],
                       rows_v.at[s], gsems[s])

    def wait_gather(s):
      pltpu.make_async_copy(tab_hbm.at[idx_v.at[pl.ds(0, _CB)]],
                            rows_v.at[s], gsems[s]).wait()

    def start_writes(k, s):
      h, off = blk_coords(k)
      for ft in range(4):
        pltpu.async_copy(
            trans_v.at[s, pl.ds(ft * _CB * 8, _CB * 8)],
            out_hbm.at[h, ft, pl.ds(off * 8, _CB * 8)], wsems[s])

    def wait_writes(s):
      for ft in range(4):
        pltpu.make_async_copy(
            trans_v.at[s, pl.ds(ft * _CB * 8, _CB * 8)],
            out_hbm.at[0, 0, pl.ds(0, _CB * 8)], wsems[s]).wait()

    def transpose(s):
      @plsc.parallel_loop(0, _CB, unroll=8)
      def _(r):
        ilo = base0 + ((r >> 7) * 1024 + (r & 127))
        plsc.store_scatter(trans_v.at[s], [ilo],
                           rows_v[s, r, pl.ds(0, 16)])
        plsc.store_scatter(trans_v.at[s], [ilo + _CB * 16],
                           rows_v[s, r, pl.ds(16, 16)])

    pltpu.async_copy(doc_hbm.at[pl.ds(base_blk * _CB, _PER_W * _CB)],
                     idx_v, isem).wait()
    for s in range(_NBUF):
      start_gather(s, s)

    def body(g, carry):
      for s in range(_NBUF):
        k = _NBUF * g + s
        wait_gather(s)

        @pl.when(g >= 1)
        def _():
          wait_writes(s)

        transpose(s)
        start_writes(k, s)

        @pl.when(g < _PER_W // _NBUF - 1)
        def _():
          start_gather(k + _NBUF, s)
      return carry

    lax.fori_loop(0, _PER_W // _NBUF, body, 0)
    for s in range(_NBUF):
      wait_writes(s)

  return gather


_gather = _make_gather()


def kernel(doc, table):
  doc_flat = doc.T.reshape(_N)
  tab_flat = lax.optimization_barrier(table.reshape(_VOCAB * _D))
  tab2 = tab_flat.reshape(_VOCAB, _D)
  out5 = _gather(doc_flat, tab2)
  out6 = out5.reshape(_HIST, 4, _BATCH // 128, 8, 128)
  return out6.transpose(2, 4, 0, 1, 3).reshape(_BATCH, _HIST, _D)
